# Initial kernel scaffold; baseline (speedup 1.0000x reference)
import jax
import jax.numpy as jnp
from jax.experimental import pallas as pl


def kernel(x, edge_index, edge_attr, batch, y, W_node_em, log_prior_node, W_edge_em, log_prior_edge, P_node_logits, P_edge_logits):
    raise NotImplementedError("write your pallas kernel here")


# speedup vs baseline: 6.6343x; 6.6343x over previous
import functools

import jax
import jax.numpy as jnp
from jax import lax
from jax.experimental import pallas as pl
from jax.experimental.pallas import tpu as pltpu
from jax.experimental.pallas import tpu_sc as plsc

_EPS = 1e-12


def _make_sc_gather(n, e):
    info = plsc.get_sparse_core_info()
    nw = info.num_cores * info.num_subcores
    assert e % (8 * nw) == 0
    e_per_w = e // nw
    mesh = plsc.VectorSubcoreMesh(core_axis_name="c", subcore_axis_name="s")

    @functools.partial(
        pl.kernel,
        mesh=mesh,
        compiler_params=pltpu.CompilerParams(needs_layout_passes=False),
        out_type=jax.ShapeDtypeStruct((e,), jnp.int32),
        scratch_types=[
            pltpu.VMEM((n,), jnp.int32),
            pltpu.VMEM((e_per_w,), jnp.int32),
            pltpu.VMEM((e_per_w,), jnp.int32),
        ],
    )
    def gather_k(y_hbm, src_hbm, out_hbm, y_v, idx_v, res_v):
        wid = lax.axis_index("s") * info.num_cores + lax.axis_index("c")
        base = wid * e_per_w
        pltpu.sync_copy(y_hbm, y_v)
        pltpu.sync_copy(src_hbm.at[pl.ds(base, e_per_w)], idx_v)

        def body(i, carry):
            idx = idx_v[pl.ds(i * 16, 16)]
            res_v[pl.ds(i * 16, 16)] = plsc.load_gather(y_v, [idx])
            return carry

        lax.fori_loop(0, e_per_w // 16, body, 0)
        pltpu.sync_copy(res_v, out_hbm.at[pl.ds(base, e_per_w)])

    return gather_k


def _softmax_rows(z):
    m = jnp.max(z, axis=1, keepdims=True)
    ez = jnp.exp(z - m)
    return ez / jnp.sum(ez, axis=1, keepdims=True)


def _node_body(nt, y_dim, x_ref, y_ref, w_ref, lp_ref, pt_ref,
               labels_ref, cll_ref, tll_ref):
    p_q = _softmax_rows(
        jnp.dot(x_ref[...], w_ref[...], preferred_element_type=jnp.float32)
        + lp_ref[...])
    p_tab = _softmax_rows(pt_ref[...])
    log_p_tab = jnp.log(p_tab + _EPS)
    oh = (y_ref[...] == lax.broadcasted_iota(jnp.int32, (nt, y_dim), 1)
          ).astype(jnp.float32)
    dn = (((1,), (1,)), ((), ()))
    ro = lax.dot_general(oh, p_tab, dn, preferred_element_type=jnp.float32)
    log_ro = lax.dot_general(oh, log_p_tab, dn,
                             preferred_element_type=jnp.float32)
    labels_ref[...] = jnp.dot(p_q, p_tab, preferred_element_type=jnp.float32)
    u = p_q * ro
    us = jnp.sum(u, axis=1, keepdims=True)
    eui = u / (us + _EPS)
    tll = jnp.sum(jnp.log(us + _EPS))
    cll = jnp.sum(eui * (jnp.log(p_q + _EPS) + log_ro))

    @pl.when(pl.program_id(0) == 0)
    def _():
        cll_ref[0, 0] = 0.0
        tll_ref[0, 0] = 0.0

    cll_ref[0, 0] += cll
    tll_ref[0, 0] += tll


def _edge_body(et, y_dim, a_ref, y_ref, w_ref, lp_ref, pt_ref,
               ncll_ref, ntll_ref, cll_ref, tll_ref):
    p_q = _softmax_rows(
        jnp.dot(a_ref[...], w_ref[...], preferred_element_type=jnp.float32)
        + lp_ref[...])
    p_tab = _softmax_rows(pt_ref[...])
    log_p_tab = jnp.log(p_tab + _EPS)
    oh = (y_ref[...] == lax.broadcasted_iota(jnp.int32, (et, y_dim), 1)
          ).astype(jnp.float32)
    dn = (((1,), (1,)), ((), ()))
    ro = lax.dot_general(oh, p_tab, dn, preferred_element_type=jnp.float32)
    log_ro = lax.dot_general(oh, log_p_tab, dn,
                             preferred_element_type=jnp.float32)
    u = p_q * ro
    us = jnp.sum(u, axis=1, keepdims=True)
    eui = u / (us + _EPS)
    tll = jnp.sum(jnp.log(us + _EPS))
    cll = jnp.sum(eui * (jnp.log(p_q + _EPS) + log_ro))

    @pl.when(pl.program_id(0) == 0)
    def _():
        cll_ref[0, 0] = ncll_ref[0, 0]
        tll_ref[0, 0] = ntll_ref[0, 0]

    cll_ref[0, 0] += cll
    tll_ref[0, 0] += tll


def _scalar_spec():
    return pl.BlockSpec((1, 1), lambda i: (0, 0), memory_space=pltpu.SMEM)


def _full_spec():
    return pl.BlockSpec(index_map=lambda i: (0, 0))


def kernel(x, edge_index, edge_attr, batch, y, W_node_em, log_prior_node,
           W_edge_em, log_prior_edge, P_node_logits, P_edge_logits):
    n, k = x.shape
    e, d_e = edge_attr.shape
    c = W_node_em.shape[1]
    ca = W_edge_em.shape[1]
    y_dim = P_node_logits.shape[1]

    src = edge_index[0].astype(jnp.int32)
    y32 = y.astype(jnp.int32)
    y_edge = _make_sc_gather(n, e)(y32, src)

    nt = 2000
    et = 16000
    assert n % nt == 0 and e % et == 0

    labels, ncll, ntll = pl.pallas_call(
        functools.partial(_node_body, nt, y_dim),
        grid=(n // nt,),
        in_specs=[
            pl.BlockSpec((nt, k), lambda i: (i, 0)),
            pl.BlockSpec((nt, 1), lambda i: (i, 0)),
            _full_spec(),
            _full_spec(),
            _full_spec(),
        ],
        out_specs=[
            pl.BlockSpec((nt, y_dim), lambda i: (i, 0)),
            _scalar_spec(),
            _scalar_spec(),
        ],
        out_shape=[
            jax.ShapeDtypeStruct((n, y_dim), jnp.float32),
            jax.ShapeDtypeStruct((1, 1), jnp.float32),
            jax.ShapeDtypeStruct((1, 1), jnp.float32),
        ],
    )(x, y32.reshape(n, 1), W_node_em, log_prior_node.reshape(1, c),
      P_node_logits)

    cll, tll = pl.pallas_call(
        functools.partial(_edge_body, et, y_dim),
        grid=(e // et,),
        in_specs=[
            pl.BlockSpec((et, d_e), lambda i: (i, 0)),
            pl.BlockSpec((et, 1), lambda i: (i, 0)),
            _full_spec(),
            _full_spec(),
            _full_spec(),
            _scalar_spec(),
            _scalar_spec(),
        ],
        out_specs=[_scalar_spec(), _scalar_spec()],
        out_shape=[
            jax.ShapeDtypeStruct((1, 1), jnp.float32),
            jax.ShapeDtypeStruct((1, 1), jnp.float32),
        ],
    )(edge_attr, y_edge.reshape(e, 1), W_edge_em,
      log_prior_edge.reshape(1, ca), P_edge_logits, ncll, ntll)

    return labels, cll.reshape(()), tll.reshape(())


# speedup vs baseline: 16.3317x; 2.4617x over previous
import functools

import jax
import jax.numpy as jnp
from jax import lax
from jax.experimental import pallas as pl
from jax.experimental.pallas import tpu as pltpu
from jax.experimental.pallas import tpu_sc as plsc

_EPS = 1e-12


def _make_sc_gather(n, e):
    info = plsc.get_sparse_core_info()
    nw = info.num_cores * info.num_subcores
    assert e % (8 * nw) == 0
    e_per_w = e // nw
    mesh = plsc.VectorSubcoreMesh(core_axis_name="c", subcore_axis_name="s")

    @functools.partial(
        pl.kernel,
        mesh=mesh,
        compiler_params=pltpu.CompilerParams(needs_layout_passes=False),
        out_type=jax.ShapeDtypeStruct((e,), jnp.int32),
        scratch_types=[
            pltpu.VMEM((n,), jnp.int32),
            pltpu.VMEM((e_per_w,), jnp.int32),
            pltpu.VMEM((e_per_w,), jnp.int32),
        ],
    )
    def gather_k(y_hbm, ei_hbm, out_hbm, y_v, idx_v, res_v):
        wid = lax.axis_index("s") * info.num_cores + lax.axis_index("c")
        base = wid * e_per_w
        pltpu.sync_copy(y_hbm, y_v)
        pltpu.sync_copy(ei_hbm.at[pl.ds(base, e_per_w)], idx_v)

        def body(i, carry):
            idx = idx_v[pl.ds(i * 16, 16)]
            res_v[pl.ds(i * 16, 16)] = plsc.load_gather(y_v, [idx])
            return carry

        lax.fori_loop(0, e_per_w // 16, body, 0)
        pltpu.sync_copy(res_v, out_hbm.at[pl.ds(base, e_per_w)])

    return gather_k


def _softmax_rows(z):
    m = jnp.max(z, axis=1, keepdims=True)
    ez = jnp.exp(z - m)
    return ez / jnp.sum(ez, axis=1, keepdims=True)


def _node_body(nt, y_dim, x_ref, y_ref, w_ref, lp_ref, pt_ref,
               labels_ref, cll_ref, tll_ref):
    p_q = _softmax_rows(
        jnp.dot(x_ref[...], w_ref[...], preferred_element_type=jnp.float32)
        + lp_ref[...])
    p_tab = _softmax_rows(pt_ref[...])
    log_p_tab = jnp.log(p_tab + _EPS)
    oh = (y_ref[...] == lax.broadcasted_iota(jnp.int32, (nt, y_dim), 1)
          ).astype(jnp.float32)
    dn = (((1,), (1,)), ((), ()))
    ro = lax.dot_general(oh, p_tab, dn, preferred_element_type=jnp.float32)
    log_ro = lax.dot_general(oh, log_p_tab, dn,
                             preferred_element_type=jnp.float32)
    labels_ref[...] = jnp.dot(p_q, p_tab, preferred_element_type=jnp.float32)
    u = p_q * ro
    us = jnp.sum(u, axis=1, keepdims=True)
    eui = u / (us + _EPS)
    tll = jnp.sum(jnp.log(us + _EPS))
    cll = jnp.sum(eui * (jnp.log(p_q + _EPS) + log_ro))

    @pl.when(pl.program_id(0) == 0)
    def _():
        cll_ref[0, 0] = 0.0
        tll_ref[0, 0] = 0.0

    cll_ref[0, 0] += cll
    tll_ref[0, 0] += tll


def _edge_body(et, y_dim, a_ref, y_ref, wt_ref, lp_ref, pt_ref,
               ncll_ref, ntll_ref, cll_ref, tll_ref):
    zt = lax.dot_general(wt_ref[...], a_ref[...], (((1,), (1,)), ((), ())),
                         preferred_element_type=jnp.float32) + lp_ref[...]
    p_tab = _softmax_rows(pt_ref[...])
    log_p_tab = jnp.log(p_tab + _EPS)
    t2_tab = p_tab * log_p_tab
    oh = (y_ref[...] == lax.broadcasted_iota(jnp.int32, (y_dim, et), 0)
          ).astype(jnp.float32)
    dn = (((1,), (0,)), ((), ()))
    r1 = lax.dot_general(p_tab, oh, dn, preferred_element_type=jnp.float32)
    r2 = lax.dot_general(t2_tab, oh, dn, preferred_element_type=jnp.float32)
    m1 = jnp.max(zt, axis=0, keepdims=True)
    e = jnp.exp(zt - m1)
    s1 = jnp.sum(e, axis=0, keepdims=True)
    t1 = e * r1
    q = jnp.sum(t1, axis=0, keepdims=True)
    s2 = jnp.sum(t1 * zt, axis=0, keepdims=True)
    s3 = jnp.sum(e * r2, axis=0, keepdims=True)
    lse = m1 + jnp.log(s1)
    tll = jnp.sum(jnp.log(q / s1 + _EPS))
    cll = jnp.sum((s2 + s3) / q - lse)

    @pl.when(pl.program_id(0) == 0)
    def _():
        cll_ref[0, 0] = ncll_ref[0, 0]
        tll_ref[0, 0] = ntll_ref[0, 0]

    cll_ref[0, 0] += cll
    tll_ref[0, 0] += tll


def _scalar_spec():
    return pl.BlockSpec((1, 1), lambda i: (0, 0), memory_space=pltpu.SMEM)


def _full_spec():
    return pl.BlockSpec(index_map=lambda i: (0, 0))


def kernel(x, edge_index, edge_attr, batch, y, W_node_em, log_prior_node,
           W_edge_em, log_prior_edge, P_node_logits, P_edge_logits):
    n, k = x.shape
    e, d_e = edge_attr.shape
    c = W_node_em.shape[1]
    ca = W_edge_em.shape[1]
    y_dim = P_node_logits.shape[1]

    y32 = y.astype(jnp.int32)
    y_edge = _make_sc_gather(n, e)(y32, edge_index.astype(jnp.int32).reshape(2 * e))

    nt = 2000
    et = 16000
    assert n % nt == 0 and e % et == 0

    labels, ncll, ntll = pl.pallas_call(
        functools.partial(_node_body, nt, y_dim),
        grid=(n // nt,),
        in_specs=[
            pl.BlockSpec((nt, k), lambda i: (i, 0)),
            pl.BlockSpec((nt, 1), lambda i: (i, 0)),
            _full_spec(),
            _full_spec(),
            _full_spec(),
        ],
        out_specs=[
            pl.BlockSpec((nt, y_dim), lambda i: (i, 0)),
            _scalar_spec(),
            _scalar_spec(),
        ],
        out_shape=[
            jax.ShapeDtypeStruct((n, y_dim), jnp.float32),
            jax.ShapeDtypeStruct((1, 1), jnp.float32),
            jax.ShapeDtypeStruct((1, 1), jnp.float32),
        ],
    )(x, y32.reshape(n, 1), W_node_em, log_prior_node.reshape(1, c),
      P_node_logits)

    cll, tll = pl.pallas_call(
        functools.partial(_edge_body, et, y_dim),
        grid=(e // et,),
        in_specs=[
            pl.BlockSpec((et, d_e), lambda i: (i, 0)),
            pl.BlockSpec((1, et), lambda i: (0, i)),
            _full_spec(),
            _full_spec(),
            _full_spec(),
            _scalar_spec(),
            _scalar_spec(),
        ],
        out_specs=[_scalar_spec(), _scalar_spec()],
        out_shape=[
            jax.ShapeDtypeStruct((1, 1), jnp.float32),
            jax.ShapeDtypeStruct((1, 1), jnp.float32),
        ],
    )(edge_attr, y_edge.reshape(1, e), W_edge_em.T,
      log_prior_edge.reshape(ca, 1), P_edge_logits, ncll, ntll)

    return labels, cll.reshape(()), tll.reshape(())


# speedup vs baseline: 17.0909x; 1.0465x over previous
import functools

import jax
import jax.numpy as jnp
from jax import lax
from jax.experimental import pallas as pl
from jax.experimental.pallas import tpu as pltpu
from jax.experimental.pallas import tpu_sc as plsc

_EPS = 1e-12


def _make_sc_gather(n, e):
    info = plsc.get_sparse_core_info()
    nw = info.num_cores * info.num_subcores
    blocks = e // 128
    active = nw
    while blocks % active:
        active -= 1
    e_per_w = e // active
    mesh = plsc.VectorSubcoreMesh(core_axis_name="c", subcore_axis_name="s")

    @functools.partial(
        pl.kernel,
        mesh=mesh,
        compiler_params=pltpu.CompilerParams(needs_layout_passes=False),
        out_type=jax.ShapeDtypeStruct((e,), jnp.int32),
        scratch_types=[
            pltpu.VMEM((n,), jnp.int32),
            pltpu.VMEM((2, e_per_w), jnp.int32),
            pltpu.VMEM((e_per_w,), jnp.int32),
        ],
    )
    def gather_k(y_hbm, ei_hbm, out_hbm, y_v, idx_v, res_v):
        wid = lax.axis_index("s") * info.num_cores + lax.axis_index("c")

        @pl.when(wid < active)
        def _():
            base = wid * e_per_w
            pltpu.sync_copy(y_hbm, y_v)
            pltpu.sync_copy(ei_hbm.at[:, pl.ds(base, e_per_w)], idx_v)

            def body(i, carry):
                idx = idx_v[0, pl.ds(i * 16, 16)]
                res_v[pl.ds(i * 16, 16)] = plsc.load_gather(y_v, [idx])
                return carry

            lax.fori_loop(0, e_per_w // 16, body, 0)
            pltpu.sync_copy(res_v, out_hbm.at[pl.ds(base, e_per_w)])

    return gather_k


def _softmax_rows(z):
    m = jnp.max(z, axis=1, keepdims=True)
    ez = jnp.exp(z - m)
    return ez / jnp.sum(ez, axis=1, keepdims=True)


def _node_body(nt, y_dim, x_ref, y_ref, w_ref, lp_ref, pt_ref,
               labels_ref, cll_ref, tll_ref):
    p_q = _softmax_rows(
        jnp.dot(x_ref[...], w_ref[...], preferred_element_type=jnp.float32)
        + lp_ref[...])
    p_tab = _softmax_rows(pt_ref[...])
    log_p_tab = jnp.log(p_tab + _EPS)
    oh = (y_ref[...] == lax.broadcasted_iota(jnp.int32, (nt, y_dim), 1)
          ).astype(jnp.float32)
    dn = (((1,), (1,)), ((), ()))
    ro = lax.dot_general(oh, p_tab, dn, preferred_element_type=jnp.float32)
    log_ro = lax.dot_general(oh, log_p_tab, dn,
                             preferred_element_type=jnp.float32)
    labels_ref[...] = jnp.dot(p_q, p_tab, preferred_element_type=jnp.float32)
    u = p_q * ro
    us = jnp.sum(u, axis=1, keepdims=True)
    eui = u / (us + _EPS)
    tll = jnp.sum(jnp.log(us + _EPS))
    cll = jnp.sum(eui * (jnp.log(p_q + _EPS) + log_ro))

    @pl.when(pl.program_id(0) == 0)
    def _():
        cll_ref[0, 0] = 0.0
        tll_ref[0, 0] = 0.0

    cll_ref[0, 0] += cll
    tll_ref[0, 0] += tll


def _edge_body(et, y_dim, a_ref, y_ref, wt_ref, lp_ref, pt_ref,
               ncll_ref, ntll_ref, cll_ref, tll_ref):
    zt = lax.dot_general(wt_ref[...], a_ref[...], (((1,), (1,)), ((), ())),
                         preferred_element_type=jnp.float32) + lp_ref[...]
    p_tab = _softmax_rows(pt_ref[...])
    log_p_tab = jnp.log(p_tab + _EPS)
    t2_tab = p_tab * log_p_tab
    oh = (y_ref[...] == lax.broadcasted_iota(jnp.int32, (y_dim, et), 0)
          ).astype(jnp.float32)
    dn = (((1,), (0,)), ((), ()))
    r1 = lax.dot_general(p_tab, oh, dn, preferred_element_type=jnp.float32)
    r2 = lax.dot_general(t2_tab, oh, dn, preferred_element_type=jnp.float32)
    m1 = jnp.max(zt, axis=0, keepdims=True)
    e = jnp.exp(zt - m1)
    s1 = jnp.sum(e, axis=0, keepdims=True)
    t1 = e * r1
    q = jnp.sum(t1, axis=0, keepdims=True)
    s2 = jnp.sum(t1 * zt, axis=0, keepdims=True)
    s3 = jnp.sum(e * r2, axis=0, keepdims=True)
    lse = m1 + jnp.log(s1)
    tll = jnp.sum(jnp.log(q / s1 + _EPS))
    cll = jnp.sum((s2 + s3) / q - lse)

    @pl.when(pl.program_id(0) == 0)
    def _():
        cll_ref[0, 0] = ncll_ref[0, 0]
        tll_ref[0, 0] = ntll_ref[0, 0]

    cll_ref[0, 0] += cll
    tll_ref[0, 0] += tll


def _scalar_spec():
    return pl.BlockSpec((1, 1), lambda i: (0, 0), memory_space=pltpu.SMEM)


def _full_spec():
    return pl.BlockSpec(index_map=lambda i: (0, 0))


def kernel(x, edge_index, edge_attr, batch, y, W_node_em, log_prior_node,
           W_edge_em, log_prior_edge, P_node_logits, P_edge_logits):
    n, k = x.shape
    e, d_e = edge_attr.shape
    c = W_node_em.shape[1]
    ca = W_edge_em.shape[1]
    y_dim = P_node_logits.shape[1]

    y32 = y.astype(jnp.int32)
    y_edge = _make_sc_gather(n, e)(y32, edge_index.astype(jnp.int32))

    nt = 2000
    et = 32000
    assert n % nt == 0 and e % et == 0

    labels, ncll, ntll = pl.pallas_call(
        functools.partial(_node_body, nt, y_dim),
        grid=(n // nt,),
        in_specs=[
            pl.BlockSpec((nt, k), lambda i: (i, 0)),
            pl.BlockSpec((nt, 1), lambda i: (i, 0)),
            _full_spec(),
            _full_spec(),
            _full_spec(),
        ],
        out_specs=[
            pl.BlockSpec((nt, y_dim), lambda i: (i, 0)),
            _scalar_spec(),
            _scalar_spec(),
        ],
        out_shape=[
            jax.ShapeDtypeStruct((n, y_dim), jnp.float32),
            jax.ShapeDtypeStruct((1, 1), jnp.float32),
            jax.ShapeDtypeStruct((1, 1), jnp.float32),
        ],
    )(x, y32.reshape(n, 1), W_node_em, log_prior_node.reshape(1, c),
      P_node_logits)

    cll, tll = pl.pallas_call(
        functools.partial(_edge_body, et, y_dim),
        grid=(e // et,),
        in_specs=[
            pl.BlockSpec((et, d_e), lambda i: (i, 0)),
            pl.BlockSpec((1, et), lambda i: (0, i)),
            _full_spec(),
            _full_spec(),
            _full_spec(),
            _scalar_spec(),
            _scalar_spec(),
        ],
        out_specs=[_scalar_spec(), _scalar_spec()],
        out_shape=[
            jax.ShapeDtypeStruct((1, 1), jnp.float32),
            jax.ShapeDtypeStruct((1, 1), jnp.float32),
        ],
    )(edge_attr, y_edge.reshape(1, e), W_edge_em.T,
      log_prior_edge.reshape(ca, 1), P_edge_logits, ncll, ntll)

    return labels, cll.reshape(()), tll.reshape(())


# speedup vs baseline: 45.8298x; 2.6815x over previous
import functools

import jax
import jax.numpy as jnp
from jax import lax
from jax.experimental import pallas as pl
from jax.experimental.pallas import tpu as pltpu
from jax.experimental.pallas import tpu_sc as plsc

_EPS = 1e-12


def _make_sc_gather(n, e):
    info = plsc.get_sparse_core_info()
    nw = info.num_cores * info.num_subcores
    blocks = e // 128
    active = nw
    while blocks % active:
        active -= 1
    e_per_w = e // active
    mesh = plsc.VectorSubcoreMesh(core_axis_name="c", subcore_axis_name="s")

    @functools.partial(
        pl.kernel,
        mesh=mesh,
        compiler_params=pltpu.CompilerParams(needs_layout_passes=False),
        out_type=jax.ShapeDtypeStruct((1, e), jnp.int32),
        scratch_types=[
            pltpu.VMEM((n,), jnp.int32),
            pltpu.VMEM((2, e_per_w), jnp.int32),
            pltpu.VMEM((e_per_w,), jnp.int32),
        ],
    )
    def gather_k(y_hbm, ei_hbm, out_hbm, y_v, idx_v, res_v):
        wid = lax.axis_index("s") * info.num_cores + lax.axis_index("c")

        @pl.when(wid < active)
        def _():
            base = wid * e_per_w
            pltpu.sync_copy(y_hbm, y_v)
            pltpu.sync_copy(ei_hbm.at[:, pl.ds(base, e_per_w)], idx_v)

            def body(i, carry):
                idx = idx_v[0, pl.ds(i * 16, 16)]
                res_v[pl.ds(i * 16, 16)] = plsc.load_gather(y_v, [idx])
                return carry

            lax.fori_loop(0, e_per_w // 16, body, 0)
            pltpu.sync_copy(res_v, out_hbm.at[0, pl.ds(base, e_per_w)])

    return gather_k


def _softmax_rows(z):
    m = jnp.max(z, axis=1, keepdims=True)
    ez = jnp.exp(z - m)
    return ez / jnp.sum(ez, axis=1, keepdims=True)


def _tables(pt):
    p_tab = _softmax_rows(pt)
    log_p_tab = jnp.log(p_tab + _EPS)
    return p_tab, p_tab * log_p_tab


def _stats(zt, y_row, p_tab, t2_tab, y_dim, width):
    oh = (y_row == lax.broadcasted_iota(jnp.int32, (y_dim, width), 0)
          ).astype(jnp.float32)
    dn = (((1,), (0,)), ((), ()))
    r1 = lax.dot_general(p_tab, oh, dn, preferred_element_type=jnp.float32)
    r2 = lax.dot_general(t2_tab, oh, dn, preferred_element_type=jnp.float32)
    m1 = jnp.max(zt, axis=0, keepdims=True)
    e = jnp.exp(zt - m1)
    s1 = jnp.sum(e, axis=0, keepdims=True)
    t1 = e * r1
    q = jnp.sum(t1, axis=0, keepdims=True)
    s2 = jnp.sum(t1 * zt, axis=0, keepdims=True)
    s3 = jnp.sum(e * r2, axis=0, keepdims=True)
    lse = m1 + jnp.log(s1)
    tll = jnp.sum(jnp.log(q / s1 + _EPS))
    cll = jnp.sum((s2 + s3) / q - lse)
    return e, s1, tll, cll


def _node_body(nt, y_dim, x_ref, y_ref, wt_ref, lp_ref, pt_ref,
               labels_ref, cll_ref, tll_ref):
    zt = lax.dot_general(wt_ref[...], x_ref[...], (((1,), (1,)), ((), ())),
                         preferred_element_type=jnp.float32) + lp_ref[...]
    p_tab, t2_tab = _tables(pt_ref[...])
    e, s1, tll, cll = _stats(zt, y_ref[...], p_tab, t2_tab, y_dim, nt)
    labels_ref[...] = lax.dot_general(
        p_tab, e, (((0,), (0,)), ((), ())),
        preferred_element_type=jnp.float32) / s1

    @pl.when(pl.program_id(0) == 0)
    def _():
        cll_ref[0, 0] = 0.0
        tll_ref[0, 0] = 0.0

    cll_ref[0, 0] += cll
    tll_ref[0, 0] += tll


def _edge_body(et, y_dim, at_ref, y_ref, wt_ref, lp_ref, pt_ref,
               ncll_ref, ntll_ref, cll_ref, tll_ref):
    zt = lax.dot_general(wt_ref[...], at_ref[...], (((1,), (0,)), ((), ())),
                         preferred_element_type=jnp.float32) + lp_ref[...]
    p_tab, t2_tab = _tables(pt_ref[...])
    _, _, tll, cll = _stats(zt, y_ref[...], p_tab, t2_tab, y_dim, et)

    @pl.when(pl.program_id(0) == 0)
    def _():
        cll_ref[0, 0] = ncll_ref[0, 0]
        tll_ref[0, 0] = ntll_ref[0, 0]

    cll_ref[0, 0] += cll
    tll_ref[0, 0] += tll


def _scalar_spec():
    return pl.BlockSpec((1, 1), lambda i: (0, 0), memory_space=pltpu.SMEM)


def _full_spec():
    return pl.BlockSpec(index_map=lambda i: (0, 0))


def kernel(x, edge_index, edge_attr, batch, y, W_node_em, log_prior_node,
           W_edge_em, log_prior_edge, P_node_logits, P_edge_logits):
    n, k = x.shape
    e, d_e = edge_attr.shape
    c = W_node_em.shape[1]
    ca = W_edge_em.shape[1]
    y_dim = P_node_logits.shape[1]

    y32 = y.astype(jnp.int32)
    y_edge = _make_sc_gather(n, e)(y32, edge_index.astype(jnp.int32))

    nt = n
    et = 32000
    assert n % nt == 0 and e % et == 0

    labels_t, ncll, ntll = pl.pallas_call(
        functools.partial(_node_body, nt, y_dim),
        grid=(n // nt,),
        in_specs=[
            pl.BlockSpec((nt, k), lambda i: (i, 0)),
            pl.BlockSpec((1, nt), lambda i: (0, i)),
            _full_spec(),
            _full_spec(),
            _full_spec(),
        ],
        out_specs=[
            pl.BlockSpec((y_dim, nt), lambda i: (0, i)),
            _scalar_spec(),
            _scalar_spec(),
        ],
        out_shape=[
            jax.ShapeDtypeStruct((y_dim, n), jnp.float32),
            jax.ShapeDtypeStruct((1, 1), jnp.float32),
            jax.ShapeDtypeStruct((1, 1), jnp.float32),
        ],
    )(x, y32.reshape(1, n), W_node_em.T, log_prior_node.reshape(c, 1),
      P_node_logits)

    cll, tll = pl.pallas_call(
        functools.partial(_edge_body, et, y_dim),
        grid=(e // et,),
        in_specs=[
            pl.BlockSpec((d_e, et), lambda i: (0, i)),
            pl.BlockSpec((1, et), lambda i: (0, i)),
            _full_spec(),
            _full_spec(),
            _full_spec(),
            _scalar_spec(),
            _scalar_spec(),
        ],
        out_specs=[_scalar_spec(), _scalar_spec()],
        out_shape=[
            jax.ShapeDtypeStruct((1, 1), jnp.float32),
            jax.ShapeDtypeStruct((1, 1), jnp.float32),
        ],
    )(edge_attr.T, y_edge, W_edge_em.T,
      log_prior_edge.reshape(ca, 1), P_edge_logits, ncll, ntll)

    return labels_t.T, cll.reshape(()), tll.reshape(())


# speedup vs baseline: 52.1394x; 1.1377x over previous
import functools

import jax
import jax.numpy as jnp
from jax import lax
from jax.experimental import pallas as pl
from jax.experimental.pallas import tpu as pltpu
from jax.experimental.pallas import tpu_sc as plsc

_EPS = 1e-12


def _make_sc_gather(n, e):
    info = plsc.get_sparse_core_info()
    nw = info.num_cores * info.num_subcores
    blocks = e // 128
    active = nw
    while blocks % active:
        active -= 1
    e_per_w = e // active
    mesh = plsc.VectorSubcoreMesh(core_axis_name="c", subcore_axis_name="s")

    @functools.partial(
        pl.kernel,
        mesh=mesh,
        compiler_params=pltpu.CompilerParams(needs_layout_passes=False),
        out_type=jax.ShapeDtypeStruct((1, e), jnp.int32),
        scratch_types=[
            pltpu.VMEM((n,), jnp.int32),
            pltpu.VMEM((2, e_per_w), jnp.int32),
            pltpu.VMEM((e_per_w,), jnp.int32),
        ],
    )
    def gather_k(y_hbm, ei_hbm, out_hbm, y_v, idx_v, res_v):
        wid = lax.axis_index("s") * info.num_cores + lax.axis_index("c")

        @pl.when(wid < active)
        def _():
            base = wid * e_per_w
            pltpu.sync_copy(y_hbm, y_v)
            pltpu.sync_copy(ei_hbm.at[:, pl.ds(base, e_per_w)], idx_v)

            def body(i, carry):
                idx = idx_v[0, pl.ds(i * 16, 16)]
                res_v[pl.ds(i * 16, 16)] = plsc.load_gather(y_v, [idx])
                return carry

            lax.fori_loop(0, e_per_w // 16, body, 0)
            pltpu.sync_copy(res_v, out_hbm.at[0, pl.ds(base, e_per_w)])

    return gather_k


def _softmax_rows(z):
    m = jnp.max(z, axis=1, keepdims=True)
    ez = jnp.exp(z - m)
    return ez / jnp.sum(ez, axis=1, keepdims=True)


def _tables(pt):
    p_tab = _softmax_rows(pt)
    log_p_tab = jnp.log(p_tab + _EPS)
    return p_tab, p_tab * log_p_tab


def _stats(zt, y_row, p_tab, t2_tab, y_dim, width):
    c = zt.shape[0]
    zc = jnp.minimum(zt, 80.0)
    e = jnp.exp(zc)
    ez = e * zc
    oh = (y_row == lax.broadcasted_iota(jnp.int32, (y_dim, width), 0)
          ).astype(jnp.float32)
    dn = (((0,), (0,)), ((), ()))
    ones = jnp.ones((1, c), jnp.float32)
    s1 = lax.dot_general(ones, e, (((1,), (0,)), ((), ())),
                         preferred_element_type=jnp.float32)
    a1 = lax.dot_general(p_tab, e, dn, preferred_element_type=jnp.float32)
    b = (lax.dot_general(p_tab, ez, dn, preferred_element_type=jnp.float32)
         + lax.dot_general(t2_tab, e, dn, preferred_element_type=jnp.float32))
    q = jnp.sum(oh * a1, axis=0, keepdims=True)
    s23 = jnp.sum(oh * b, axis=0, keepdims=True)
    sum_log_s1 = jnp.sum(jnp.log(s1))
    tll = jnp.sum(jnp.log(q)) - sum_log_s1
    cll = jnp.sum(s23 / q) - sum_log_s1
    return a1, s1, tll, cll


def _node_body(nt, y_dim, x_ref, y_ref, wt_ref, lp_ref, pt_ref,
               labels_ref, cll_ref, tll_ref):
    zt = lax.dot_general(wt_ref[...], x_ref[...], (((1,), (1,)), ((), ())),
                         preferred_element_type=jnp.float32) + lp_ref[...]
    p_tab, t2_tab = _tables(pt_ref[...])
    a1, s1, tll, cll = _stats(zt, y_ref[...], p_tab, t2_tab, y_dim, nt)
    labels_ref[...] = a1 / s1

    @pl.when(pl.program_id(0) == 0)
    def _():
        cll_ref[0, 0] = 0.0
        tll_ref[0, 0] = 0.0

    cll_ref[0, 0] += cll
    tll_ref[0, 0] += tll


def _edge_body(et, y_dim, at_ref, y_ref, wt_ref, lp_ref, pt_ref,
               ncll_ref, ntll_ref, cll_ref, tll_ref):
    zt = lax.dot_general(wt_ref[...], at_ref[...], (((1,), (0,)), ((), ())),
                         preferred_element_type=jnp.float32) + lp_ref[...]
    p_tab, t2_tab = _tables(pt_ref[...])
    _, _, tll, cll = _stats(zt, y_ref[...], p_tab, t2_tab, y_dim, et)

    @pl.when(pl.program_id(0) == 0)
    def _():
        cll_ref[0, 0] = ncll_ref[0, 0]
        tll_ref[0, 0] = ntll_ref[0, 0]

    cll_ref[0, 0] += cll
    tll_ref[0, 0] += tll


def _scalar_spec():
    return pl.BlockSpec((1, 1), lambda i: (0, 0), memory_space=pltpu.SMEM)


def _full_spec():
    return pl.BlockSpec(index_map=lambda i: (0, 0))


def kernel(x, edge_index, edge_attr, batch, y, W_node_em, log_prior_node,
           W_edge_em, log_prior_edge, P_node_logits, P_edge_logits):
    n, k = x.shape
    e, d_e = edge_attr.shape
    c = W_node_em.shape[1]
    ca = W_edge_em.shape[1]
    y_dim = P_node_logits.shape[1]

    y32 = y.astype(jnp.int32)
    y_edge = _make_sc_gather(n, e)(y32, edge_index.astype(jnp.int32))

    nt = n
    et = 32000
    assert n % nt == 0 and e % et == 0

    labels_t, ncll, ntll = pl.pallas_call(
        functools.partial(_node_body, nt, y_dim),
        grid=(n // nt,),
        in_specs=[
            pl.BlockSpec((nt, k), lambda i: (i, 0)),
            pl.BlockSpec((1, nt), lambda i: (0, i)),
            _full_spec(),
            _full_spec(),
            _full_spec(),
        ],
        out_specs=[
            pl.BlockSpec((y_dim, nt), lambda i: (0, i)),
            _scalar_spec(),
            _scalar_spec(),
        ],
        out_shape=[
            jax.ShapeDtypeStruct((y_dim, n), jnp.float32),
            jax.ShapeDtypeStruct((1, 1), jnp.float32),
            jax.ShapeDtypeStruct((1, 1), jnp.float32),
        ],
    )(x, y32.reshape(1, n), W_node_em.T, log_prior_node.reshape(c, 1),
      P_node_logits)

    cll, tll = pl.pallas_call(
        functools.partial(_edge_body, et, y_dim),
        grid=(e // et,),
        in_specs=[
            pl.BlockSpec((d_e, et), lambda i: (0, i)),
            pl.BlockSpec((1, et), lambda i: (0, i)),
            _full_spec(),
            _full_spec(),
            _full_spec(),
            _scalar_spec(),
            _scalar_spec(),
        ],
        out_specs=[_scalar_spec(), _scalar_spec()],
        out_shape=[
            jax.ShapeDtypeStruct((1, 1), jnp.float32),
            jax.ShapeDtypeStruct((1, 1), jnp.float32),
        ],
    )(edge_attr.T, y_edge, W_edge_em.T,
      log_prior_edge.reshape(ca, 1), P_edge_logits, ncll, ntll)

    return labels_t.T, cll.reshape(()), tll.reshape(())


# speedup vs baseline: 56.7465x; 1.0884x over previous
import functools

import jax
import jax.numpy as jnp
from jax import lax
from jax.experimental import pallas as pl
from jax.experimental.pallas import tpu as pltpu
from jax.experimental.pallas import tpu_sc as plsc

_EPS = 1e-12


def _make_sc_gather(n, e):
    info = plsc.get_sparse_core_info()
    nw = info.num_cores * info.num_subcores
    blocks = e // 128
    active = nw
    while blocks % active:
        active -= 1
    e_per_w = e // active
    mesh = plsc.VectorSubcoreMesh(core_axis_name="c", subcore_axis_name="s")

    @functools.partial(
        pl.kernel,
        mesh=mesh,
        compiler_params=pltpu.CompilerParams(needs_layout_passes=False),
        out_type=jax.ShapeDtypeStruct((1, e), jnp.int32),
        scratch_types=[
            pltpu.VMEM((n,), jnp.int32),
            pltpu.VMEM((2, e_per_w), jnp.int32),
            pltpu.VMEM((e_per_w,), jnp.int32),
        ],
    )
    def gather_k(y_hbm, ei_hbm, out_hbm, y_v, idx_v, res_v):
        wid = lax.axis_index("s") * info.num_cores + lax.axis_index("c")

        @pl.when(wid < active)
        def _():
            base = wid * e_per_w
            pltpu.sync_copy(y_hbm, y_v)
            pltpu.sync_copy(ei_hbm.at[:, pl.ds(base, e_per_w)], idx_v)

            @plsc.parallel_loop(0, e_per_w, 16, unroll=8)
            def body(i):
                idx = idx_v[0, pl.ds(i, 16)]
                res_v[pl.ds(i, 16)] = plsc.load_gather(y_v, [idx])
            pltpu.sync_copy(res_v, out_hbm.at[0, pl.ds(base, e_per_w)])

    return gather_k


def _softmax_rows(z):
    m = jnp.max(z, axis=1, keepdims=True)
    ez = jnp.exp(z - m)
    return ez / jnp.sum(ez, axis=1, keepdims=True)


def _tables(pt):
    p_tab = _softmax_rows(pt)
    log_p_tab = jnp.log(p_tab + _EPS)
    return p_tab, p_tab * log_p_tab


def _stats(zt, y_row, p_tab, t2_tab, y_dim, width):
    c = zt.shape[0]
    zc = jnp.minimum(zt, 80.0)
    e = jnp.exp(zc)
    ez = e * zc
    oh = (y_row == lax.broadcasted_iota(jnp.int32, (y_dim, width), 0)
          ).astype(jnp.float32)
    dn = (((0,), (0,)), ((), ()))
    ones = jnp.ones((1, c), jnp.float32)
    s1 = lax.dot_general(ones, e, (((1,), (0,)), ((), ())),
                         preferred_element_type=jnp.float32)
    a1 = lax.dot_general(p_tab, e, dn, preferred_element_type=jnp.float32)
    b = (lax.dot_general(p_tab, ez, dn, preferred_element_type=jnp.float32)
         + lax.dot_general(t2_tab, e, dn, preferred_element_type=jnp.float32))
    q = jnp.sum(oh * a1, axis=0, keepdims=True)
    s23 = jnp.sum(oh * b, axis=0, keepdims=True)
    sum_log_s1 = jnp.sum(jnp.log(s1))
    tll = jnp.sum(jnp.log(q)) - sum_log_s1
    cll = jnp.sum(s23 / q) - sum_log_s1
    return a1, s1, tll, cll


def _node_body(nt, y_dim, x_ref, y_ref, wt_ref, lp_ref, pt_ref,
               labels_ref, cll_ref, tll_ref):
    zt = lax.dot_general(wt_ref[...], x_ref[...], (((1,), (1,)), ((), ())),
                         preferred_element_type=jnp.float32) + lp_ref[...]
    p_tab, t2_tab = _tables(pt_ref[...])
    a1, s1, tll, cll = _stats(zt, y_ref[...], p_tab, t2_tab, y_dim, nt)
    labels_ref[...] = a1 / s1

    @pl.when(pl.program_id(0) == 0)
    def _():
        cll_ref[0, 0] = 0.0
        tll_ref[0, 0] = 0.0

    cll_ref[0, 0] += cll
    tll_ref[0, 0] += tll


def _edge_body(et, y_dim, at_ref, y_ref, wt_ref, lp_ref, pt_ref,
               ncll_ref, ntll_ref, cll_ref, tll_ref):
    zt = lax.dot_general(wt_ref[...], at_ref[...], (((1,), (0,)), ((), ())),
                         preferred_element_type=jnp.float32) + lp_ref[...]
    p_tab, t2_tab = _tables(pt_ref[...])
    _, _, tll, cll = _stats(zt, y_ref[...], p_tab, t2_tab, y_dim, et)

    @pl.when(pl.program_id(0) == 0)
    def _():
        cll_ref[0, 0] = ncll_ref[0, 0]
        tll_ref[0, 0] = ntll_ref[0, 0]

    cll_ref[0, 0] += cll
    tll_ref[0, 0] += tll


def _scalar_spec():
    return pl.BlockSpec((1, 1), lambda i: (0, 0), memory_space=pltpu.SMEM)


def _full_spec():
    return pl.BlockSpec(index_map=lambda i: (0, 0))


def kernel(x, edge_index, edge_attr, batch, y, W_node_em, log_prior_node,
           W_edge_em, log_prior_edge, P_node_logits, P_edge_logits):
    n, k = x.shape
    e, d_e = edge_attr.shape
    c = W_node_em.shape[1]
    ca = W_edge_em.shape[1]
    y_dim = P_node_logits.shape[1]

    y32 = y.astype(jnp.int32)
    y_edge = _make_sc_gather(n, e)(y32, edge_index.astype(jnp.int32))

    nt = n
    et = 64000
    assert n % nt == 0 and e % et == 0

    labels_t, ncll, ntll = pl.pallas_call(
        functools.partial(_node_body, nt, y_dim),
        grid=(n // nt,),
        in_specs=[
            pl.BlockSpec((nt, k), lambda i: (i, 0)),
            pl.BlockSpec((1, nt), lambda i: (0, i)),
            _full_spec(),
            _full_spec(),
            _full_spec(),
        ],
        out_specs=[
            pl.BlockSpec((y_dim, nt), lambda i: (0, i)),
            _scalar_spec(),
            _scalar_spec(),
        ],
        out_shape=[
            jax.ShapeDtypeStruct((y_dim, n), jnp.float32),
            jax.ShapeDtypeStruct((1, 1), jnp.float32),
            jax.ShapeDtypeStruct((1, 1), jnp.float32),
        ],
    )(x, y32.reshape(1, n), W_node_em.T, log_prior_node.reshape(c, 1),
      P_node_logits)

    cll, tll = pl.pallas_call(
        functools.partial(_edge_body, et, y_dim),
        grid=(e // et,),
        in_specs=[
            pl.BlockSpec((d_e, et), lambda i: (0, i)),
            pl.BlockSpec((1, et), lambda i: (0, i)),
            _full_spec(),
            _full_spec(),
            _full_spec(),
            _scalar_spec(),
            _scalar_spec(),
        ],
        out_specs=[_scalar_spec(), _scalar_spec()],
        out_shape=[
            jax.ShapeDtypeStruct((1, 1), jnp.float32),
            jax.ShapeDtypeStruct((1, 1), jnp.float32),
        ],
    )(edge_attr.T, y_edge, W_edge_em.T,
      log_prior_edge.reshape(ca, 1), P_edge_logits, ncll, ntll)

    return labels_t.T, cll.reshape(()), tll.reshape(())


# speedup vs baseline: 57.4017x; 1.0115x over previous
import functools

import jax
import jax.numpy as jnp
from jax import lax
from jax.experimental import pallas as pl
from jax.experimental.pallas import tpu as pltpu
from jax.experimental.pallas import tpu_sc as plsc

_EPS = 1e-12
_SHIFT = 32.0


def _make_sc_gather(n, e):
    info = plsc.get_sparse_core_info()
    nw = info.num_cores * info.num_subcores
    blocks = e // 128
    active = nw
    while blocks % active:
        active -= 1
    e_per_w = e // active
    mesh = plsc.VectorSubcoreMesh(core_axis_name="c", subcore_axis_name="s")

    @functools.partial(
        pl.kernel,
        mesh=mesh,
        compiler_params=pltpu.CompilerParams(needs_layout_passes=False),
        out_type=jax.ShapeDtypeStruct((1, e), jnp.int32),
        scratch_types=[
            pltpu.VMEM((n,), jnp.int32),
            pltpu.VMEM((2, e_per_w), jnp.int32),
            pltpu.VMEM((e_per_w,), jnp.int32),
        ],
    )
    def gather_k(y_hbm, ei_hbm, out_hbm, y_v, idx_v, res_v):
        wid = lax.axis_index("s") * info.num_cores + lax.axis_index("c")

        @pl.when(wid < active)
        def _():
            base = wid * e_per_w
            pltpu.sync_copy(y_hbm, y_v)
            pltpu.sync_copy(ei_hbm.at[:, pl.ds(base, e_per_w)], idx_v)

            @plsc.parallel_loop(0, e_per_w, 16, unroll=8)
            def body(i):
                idx = idx_v[0, pl.ds(i, 16)]
                res_v[pl.ds(i, 16)] = plsc.load_gather(y_v, [idx])
            pltpu.sync_copy(res_v, out_hbm.at[0, pl.ds(base, e_per_w)])

    return gather_k


def _softmax_rows(z):
    m = jnp.max(z, axis=1, keepdims=True)
    ez = jnp.exp(z - m)
    return ez / jnp.sum(ez, axis=1, keepdims=True)


def _tables(pt):
    p_tab = _softmax_rows(pt)
    log_p_tab = jnp.log(p_tab + _EPS)
    return p_tab, p_tab * log_p_tab


def _stats(zt, y_row, p_tab, t2_tab, y_dim, width, low_precision=False):
    c = zt.shape[0]
    e = jnp.exp(zt)
    ez = e * zt
    if low_precision:
        mm_dtype = jnp.bfloat16
        e = e.astype(mm_dtype)
        ez = ez.astype(mm_dtype)
        p_tab_mm = p_tab.astype(mm_dtype)
        t2_tab_mm = t2_tab.astype(mm_dtype)
    else:
        mm_dtype = jnp.float32
        p_tab_mm, t2_tab_mm = p_tab, t2_tab
    oh = y_row == lax.broadcasted_iota(jnp.int32, (y_dim, width), 0)
    dn = (((0,), (0,)), ((), ()))
    ones = jnp.ones((1, c), mm_dtype)
    s1 = lax.dot_general(ones, e, (((1,), (0,)), ((), ())),
                         preferred_element_type=jnp.float32)
    a1 = lax.dot_general(p_tab_mm, e, dn, preferred_element_type=jnp.float32)
    b = (lax.dot_general(p_tab_mm, ez, dn, preferred_element_type=jnp.float32)
         + lax.dot_general(t2_tab_mm, e, dn,
                           preferred_element_type=jnp.float32))
    zero = jnp.zeros((), jnp.float32)
    q = jnp.sum(jnp.where(oh, a1, zero), axis=0, keepdims=True)
    s23 = jnp.sum(jnp.where(oh, b, zero), axis=0, keepdims=True)
    sum_log_s1 = jnp.sum(jnp.log(s1))
    tll = jnp.sum(jnp.log(q)) - sum_log_s1
    cll = jnp.sum(s23 / q) - sum_log_s1
    return a1, s1, tll, cll


def _node_body(nt, y_dim, x_ref, y_ref, wt_ref, lp_ref, pt_ref,
               labels_ref, cll_ref, tll_ref):
    zt = lax.dot_general(wt_ref[...], x_ref[...], (((1,), (1,)), ((), ())),
                         preferred_element_type=jnp.float32) + (lp_ref[...]
                                                                - _SHIFT)
    p_tab, t2_tab = _tables(pt_ref[...])
    a1, s1, tll, cll = _stats(zt, y_ref[...], p_tab, t2_tab, y_dim, nt)
    labels_ref[...] = a1 / s1

    @pl.when(pl.program_id(0) == 0)
    def _():
        cll_ref[0, 0] = 0.0
        tll_ref[0, 0] = 0.0

    cll_ref[0, 0] += cll
    tll_ref[0, 0] += tll


def _edge_body(et, y_dim, at_ref, y_ref, wt_ref, lp_ref, pt_ref,
               ncll_ref, ntll_ref, cll_ref, tll_ref):
    zt = lax.dot_general(wt_ref[...], at_ref[...], (((1,), (0,)), ((), ())),
                         preferred_element_type=jnp.float32) + (lp_ref[...]
                                                                - _SHIFT)
    p_tab, t2_tab = _tables(pt_ref[...])
    _, _, tll, cll = _stats(zt, y_ref[...], p_tab, t2_tab, y_dim, et, low_precision=True)

    @pl.when(pl.program_id(0) == 0)
    def _():
        cll_ref[0, 0] = ncll_ref[0, 0]
        tll_ref[0, 0] = ntll_ref[0, 0]

    cll_ref[0, 0] += cll
    tll_ref[0, 0] += tll


def _scalar_spec():
    return pl.BlockSpec((1, 1), lambda i: (0, 0), memory_space=pltpu.SMEM)


def _full_spec():
    return pl.BlockSpec(index_map=lambda i: (0, 0))


def kernel(x, edge_index, edge_attr, batch, y, W_node_em, log_prior_node,
           W_edge_em, log_prior_edge, P_node_logits, P_edge_logits):
    n, k = x.shape
    e, d_e = edge_attr.shape
    c = W_node_em.shape[1]
    ca = W_edge_em.shape[1]
    y_dim = P_node_logits.shape[1]

    y32 = y.astype(jnp.int32)
    y_edge = _make_sc_gather(n, e)(y32, edge_index.astype(jnp.int32))

    nt = n
    et = 32000
    assert n % nt == 0 and e % et == 0

    labels_t, ncll, ntll = pl.pallas_call(
        functools.partial(_node_body, nt, y_dim),
        grid=(n // nt,),
        in_specs=[
            pl.BlockSpec((nt, k), lambda i: (i, 0)),
            pl.BlockSpec((1, nt), lambda i: (0, i)),
            _full_spec(),
            _full_spec(),
            _full_spec(),
        ],
        out_specs=[
            pl.BlockSpec((y_dim, nt), lambda i: (0, i)),
            _scalar_spec(),
            _scalar_spec(),
        ],
        out_shape=[
            jax.ShapeDtypeStruct((y_dim, n), jnp.float32),
            jax.ShapeDtypeStruct((1, 1), jnp.float32),
            jax.ShapeDtypeStruct((1, 1), jnp.float32),
        ],
    )(x, y32.reshape(1, n), W_node_em.T, log_prior_node.reshape(c, 1),
      P_node_logits)

    cll, tll = pl.pallas_call(
        functools.partial(_edge_body, et, y_dim),
        grid=(e // et,),
        in_specs=[
            pl.BlockSpec((d_e, et), lambda i: (0, i)),
            pl.BlockSpec((1, et), lambda i: (0, i)),
            _full_spec(),
            _full_spec(),
            _full_spec(),
            _scalar_spec(),
            _scalar_spec(),
        ],
        out_specs=[_scalar_spec(), _scalar_spec()],
        out_shape=[
            jax.ShapeDtypeStruct((1, 1), jnp.float32),
            jax.ShapeDtypeStruct((1, 1), jnp.float32),
        ],
    )(edge_attr.T, y_edge, W_edge_em.T,
      log_prior_edge.reshape(ca, 1), P_edge_logits, ncll, ntll)

    return labels_t.T, cll.reshape(()), tll.reshape(())
